# baseline (speedup 1.0000x reference)
import functools

import jax
import jax.numpy as jnp
from jax.experimental import pallas as pl
from jax.experimental.pallas import tpu as pltpu


def _upgate_body(x_ref, wu_ref, wg_ref, z_ref):
    x = x_ref[...]
    up = jnp.dot(x, wu_ref[...], preferred_element_type=jnp.float32)
    gate = jnp.dot(x, wg_ref[...], preferred_element_type=jnp.float32)
    z_ref[...] = (up * gate * jax.lax.logistic(gate)).astype(jnp.bfloat16)


def _down_body(bn, z_ref, wd_ref, o_ref):
    j = pl.program_id(1)
    o_ref[...] = jnp.dot(z_ref[...], wd_ref[:, pl.ds(j * bn, bn)],
                         preferred_element_type=jnp.float32)


@functools.partial(jax.jit, static_argnames=("bm", "bnz", "bmd", "bn"))
def _fused_swiglu(xf, wu, wg, wd, bm=4096, bnz=256, bmd=512, bn=512):
    m, d_model = xf.shape
    d_ff = wu.shape[1]
    z = pl.pallas_call(
        _upgate_body,
        grid=(m // bm, d_ff // bnz),
        in_specs=[
            pl.BlockSpec((bm, d_model), lambda i, j: (i, 0)),
            pl.BlockSpec((d_model, bnz), lambda i, j: (0, j)),
            pl.BlockSpec((d_model, bnz), lambda i, j: (0, j)),
        ],
        out_specs=pl.BlockSpec((bm, bnz), lambda i, j: (i, j)),
        out_shape=jax.ShapeDtypeStruct((m, d_ff), jnp.bfloat16),
        compiler_params=pltpu.CompilerParams(
            dimension_semantics=("parallel", "arbitrary"),
        ),
    )(xf, wu, wg)
    return pl.pallas_call(
        functools.partial(_down_body, bn),
        grid=(m // bmd, d_model // bn),
        in_specs=[
            pl.BlockSpec((bmd, d_ff), lambda i, j: (i, 0)),
            pl.BlockSpec((d_ff, d_model), lambda i, j: (0, 0)),
        ],
        out_specs=pl.BlockSpec((bmd, bn), lambda i, j: (i, j)),
        out_shape=jax.ShapeDtypeStruct((m, d_model), jnp.float32),
        compiler_params=pltpu.CompilerParams(
            dimension_semantics=("parallel", "arbitrary"),
        ),
    )(z, wd)


def kernel(x, Wupt, Wgatet, Wdownt):
    b, s, d_model = x.shape
    xf = x.reshape(b * s, d_model).astype(jnp.bfloat16)
    out = _fused_swiglu(
        xf,
        Wupt.astype(jnp.bfloat16),
        Wgatet.astype(jnp.bfloat16),
        Wdownt.astype(jnp.bfloat16),
    )
    return out.reshape(b, s, d_model)


# speedup vs baseline: 1.1208x; 1.1208x over previous
import functools

import jax
import jax.numpy as jnp
from jax.experimental import pallas as pl
from jax.experimental.pallas import tpu as pltpu


def _upgate_body(x_ref, wu_ref, wg_ref, z_ref):
    x = x_ref[...]
    wu = wu_ref[...].astype(jnp.bfloat16)
    wg = wg_ref[...].astype(jnp.bfloat16)
    up = jnp.dot(x, wu, preferred_element_type=jnp.float32)
    gate = jnp.dot(x, wg, preferred_element_type=jnp.float32)
    z_ref[...] = (up * gate * jax.lax.logistic(gate)).astype(jnp.bfloat16)


def _down_body(bn, z_ref, wd_ref, o_ref):
    j = pl.program_id(1)
    o_ref[...] = jnp.dot(z_ref[...], wd_ref[:, pl.ds(j * bn, bn)],
                         preferred_element_type=jnp.float32)


@functools.partial(jax.jit, static_argnames=("bm", "bnz", "bmd", "bn"))
def _fused_swiglu(xf, wu, wg, wd, bm=4096, bnz=256, bmd=512, bn=512):
    m, d_model = xf.shape
    d_ff = wu.shape[1]
    z = pl.pallas_call(
        _upgate_body,
        grid=(m // bm, d_ff // bnz),
        in_specs=[
            pl.BlockSpec((bm, d_model), lambda i, j: (i, 0)),
            pl.BlockSpec((d_model, bnz), lambda i, j: (0, j)),
            pl.BlockSpec((d_model, bnz), lambda i, j: (0, j)),
        ],
        out_specs=pl.BlockSpec((bm, bnz), lambda i, j: (i, j)),
        out_shape=jax.ShapeDtypeStruct((m, d_ff), jnp.bfloat16),
        compiler_params=pltpu.CompilerParams(
            dimension_semantics=("parallel", "arbitrary"),
        ),
    )(xf, wu, wg)
    return pl.pallas_call(
        functools.partial(_down_body, bn),
        grid=(m // bmd, d_model // bn),
        in_specs=[
            pl.BlockSpec((bmd, d_ff), lambda i, j: (i, 0)),
            pl.BlockSpec((d_ff, d_model), lambda i, j: (0, 0)),
        ],
        out_specs=pl.BlockSpec((bmd, bn), lambda i, j: (i, j)),
        out_shape=jax.ShapeDtypeStruct((m, d_model), jnp.float32),
        compiler_params=pltpu.CompilerParams(
            dimension_semantics=("parallel", "arbitrary"),
        ),
    )(z, wd)


def kernel(x, Wupt, Wgatet, Wdownt):
    b, s, d_model = x.shape
    xf = x.reshape(b * s, d_model).astype(jnp.bfloat16)
    out = _fused_swiglu(xf, Wupt, Wgatet, Wdownt.astype(jnp.bfloat16))
    return out.reshape(b, s, d_model)


# speedup vs baseline: 1.1859x; 1.0581x over previous
import functools

import jax
import jax.numpy as jnp
from jax.experimental import pallas as pl
from jax.experimental.pallas import tpu as pltpu


def _upgate_body(x_ref, wu_ref, wg_ref, wd_ref, z_ref, wdb_ref):
    x = x_ref[...]
    wu = wu_ref[...].astype(jnp.bfloat16)
    wg = wg_ref[...].astype(jnp.bfloat16)
    up = jnp.dot(x, wu, preferred_element_type=jnp.float32)
    gate = jnp.dot(x, wg, preferred_element_type=jnp.float32)
    z_ref[...] = (up * gate * jax.lax.logistic(gate)).astype(jnp.bfloat16)
    wdb_ref[...] = wd_ref[...].astype(jnp.bfloat16)


def _down_body(bn, z_ref, wd_ref, o_ref):
    j = pl.program_id(1)
    o_ref[...] = jnp.dot(z_ref[...], wd_ref[:, pl.ds(j * bn, bn)],
                         preferred_element_type=jnp.float32)


@functools.partial(jax.jit, static_argnames=("bm", "bnz", "bmd", "bn"))
def _fused_swiglu(xf, wu, wg, wd, bm=4096, bnz=256, bmd=512, bn=512):
    m, d_model = xf.shape
    d_ff = wu.shape[1]
    z, wdb = pl.pallas_call(
        _upgate_body,
        grid=(m // bm, d_ff // bnz),
        in_specs=[
            pl.BlockSpec((bm, d_model), lambda i, j: (i, 0)),
            pl.BlockSpec((d_model, bnz), lambda i, j: (0, j)),
            pl.BlockSpec((d_model, bnz), lambda i, j: (0, j)),
            pl.BlockSpec((bnz, d_model), lambda i, j: (j, 0)),
        ],
        out_specs=[
            pl.BlockSpec((bm, bnz), lambda i, j: (i, j)),
            pl.BlockSpec((bnz, d_model), lambda i, j: (j, 0)),
        ],
        out_shape=[
            jax.ShapeDtypeStruct((m, d_ff), jnp.bfloat16),
            jax.ShapeDtypeStruct((d_ff, d_model), jnp.bfloat16),
        ],
        compiler_params=pltpu.CompilerParams(
            dimension_semantics=("parallel", "arbitrary"),
        ),
    )(xf, wu, wg, wd)
    return pl.pallas_call(
        functools.partial(_down_body, bn),
        grid=(m // bmd, d_model // bn),
        in_specs=[
            pl.BlockSpec((bmd, d_ff), lambda i, j: (i, 0)),
            pl.BlockSpec((d_ff, d_model), lambda i, j: (0, 0)),
        ],
        out_specs=pl.BlockSpec((bmd, bn), lambda i, j: (i, j)),
        out_shape=jax.ShapeDtypeStruct((m, d_model), jnp.float32),
        compiler_params=pltpu.CompilerParams(
            dimension_semantics=("parallel", "arbitrary"),
        ),
    )(z, wdb)


def kernel(x, Wupt, Wgatet, Wdownt):
    b, s, d_model = x.shape
    xf = x.reshape(b * s, d_model).astype(jnp.bfloat16)
    out = _fused_swiglu(xf, Wupt, Wgatet, Wdownt)
    return out.reshape(b, s, d_model)
